# baseline (speedup 1.0000x reference)
import functools

import jax
import jax.numpy as jnp
from jax.experimental import pallas as pl
from jax.experimental.pallas import tpu as pltpu

_SKIP = 0.5

_BI = 512
_BJ = 1024
_BIA = 512

_BF = jnp.bfloat16


def _dot(a, b):
    return jax.lax.dot_general(
        a, b, (((1,), (0,)), ((), ())),
        preferred_element_type=jnp.float32)


def _dot_sm(a, b):
    return jax.lax.dot_general(
        a.astype(_BF), b.astype(_BF), (((1,), (0,)), ((), ())),
        preferred_element_type=jnp.float32)


def _dinv_of(deg):
    return jnp.where(deg > 0, 1.0 / jnp.sqrt(deg + 1e-12), 0.0)


def _dinv0_kernel(a_ref, dinv_ref, abf_ref, acc_ref, *, nj, bj, d):
    j = pl.program_id(1)

    @pl.when(j == 0)
    def _():
        acc_ref[...] = jnp.zeros_like(acc_ref)

    abf = a_ref[...].astype(_BF)
    abf_ref[...] = abf
    ones = jnp.ones((bj, d), dtype=_BF)
    acc_ref[...] += _dot(abf, ones)

    @pl.when(j == nj - 1)
    def _():
        dinv_ref[...] = _dinv_of(acc_ref[:, :1])


def _xw_kernel(x_ref, w1_ref, wi_ref, dinv0_ref, xw1_ref, m0_ref):
    xw1 = _dot_sm(x_ref[...], w1_ref[...])
    xwi = _dot_sm(x_ref[...], wi_ref[...])
    xw1_ref[...] = xw1.astype(_BF)
    d0 = dinv0_ref[...]
    m0_ref[:, : xw1.shape[1]] = (d0 * xw1).astype(_BF)
    m0_ref[:, xw1.shape[1]:] = (d0 * xwi).astype(_BF)


def _init_kernel(ah_ref, m0_ref, dinv0_ref, w2_ref, wg_ref,
                 hn_ref, hnT_ref, g_ref, gv0_ref,
                 acc_ref, *, nj, bj, d):
    j = pl.program_id(1)

    @pl.when(j == 0)
    def _():
        acc_ref[...] = jnp.zeros_like(acc_ref)

    acc_ref[...] += _dot(ah_ref[...], m0_ref[pl.ds(j * bj, bj), :])

    @pl.when(j == nj - 1)
    def _():
        d0 = dinv0_ref[...]
        p = acc_ref[...] * d0
        h_c0 = jnp.maximum(p[:, :d], 0.0)
        h = jnp.maximum(p[:, d:], 0.0)
        nrm = jnp.sqrt(jnp.sum(h * h, axis=1, keepdims=True)) + 1e-8
        hn = (h / nrm).astype(_BF)
        hn_ref[...] = hn
        hnT_ref[...] = hn.T
        g = _dot_sm(h, wg_ref[...])
        g_ref[...] = g.astype(_BF)
        gv0_ref[:, :d] = (d0 * g).astype(_BF)
        v0 = d0 * _dot_sm(h_c0, w2_ref[...])
        gv0_ref[:, d:] = v0.astype(_BF)


def _deg_kernel(hn_ref, hnT_ref, g_ref, xw1_ref, dinv_ref, mcat_ref, s_ref,
                acc_ref, *, nj, bj, d):
    j = pl.program_id(1)

    @pl.when(j == 0)
    def _():
        acc_ref[...] = jnp.zeros_like(acc_ref)

    s = jnp.maximum(_dot(hn_ref[...], hnT_ref[:, pl.ds(j * bj, bj)]), 0.0)
    s_ref[...] = s.astype(_BF)
    acc_ref[...] += jnp.sum(s, axis=1, keepdims=True)

    @pl.when(j == nj - 1)
    def _():
        dinv = _dinv_of(acc_ref[...])
        dinv_ref[...] = dinv
        mcat_ref[:, :d] = (dinv * g_ref[...].astype(jnp.float32)).astype(_BF)
        mcat_ref[:, d:] = (dinv * xw1_ref[...].astype(jnp.float32)).astype(_BF)


def _fused_kernel(s_ref, mcat_ref, ah_ref, gv0_ref, dinv_ref,
                  dinv0_ref, w2_ref, wgn_ref, mask_ref,
                  hn2_ref, hnT2_ref, g2_ref, v_ref, jk0_ref,
                  accs_ref, accc_ref, *, nj, bj, d):
    j = pl.program_id(1)

    @pl.when(j == 0)
    def _():
        accs_ref[...] = jnp.zeros_like(accs_ref)
        accc_ref[...] = jnp.zeros_like(accc_ref)

    accs_ref[...] += _dot(s_ref[...], mcat_ref[pl.ds(j * bj, bj), :])
    accc_ref[...] += _dot(ah_ref[...], gv0_ref[pl.ds(j * bj, bj), :])

    @pl.when(j == nj - 1)
    def _():
        dinv = dinv_ref[...]
        d0 = dinv0_ref[...]
        h_new = jnp.maximum(
            _SKIP * d0 * accc_ref[:, :d] + (1.0 - _SKIP) * dinv * accs_ref[:, :d],
            0.0)
        h2 = jnp.maximum(dinv * accs_ref[:, d:], 0.0)
        jk0_ref[...] = mask_ref[...] * d0 * accc_ref[:, d:]
        v_ref[...] = (dinv * _dot_sm(h2, w2_ref[...])).astype(_BF)
        nrm = jnp.sqrt(jnp.sum(h_new * h_new, axis=1, keepdims=True)) + 1e-8
        hn2 = (h_new / nrm).astype(_BF)
        hn2_ref[...] = hn2
        hnT2_ref[...] = hn2.T
        g2_ref[...] = _dot_sm(h_new, wgn_ref[...]).astype(_BF)


def _fused2_kernel(s_ref, m_ref, dinv_ref, w2_ref, v_ref,
                   acc_ref, *, nj, bj):
    j = pl.program_id(1)

    @pl.when(j == 0)
    def _():
        acc_ref[...] = jnp.zeros_like(acc_ref)

    acc_ref[...] += _dot(s_ref[...], m_ref[pl.ds(j * bj, bj), :])

    @pl.when(j == nj - 1)
    def _():
        dinv = dinv_ref[...]
        h2 = jnp.maximum(dinv * acc_ref[...], 0.0)
        v_ref[...] = (dinv * _dot_sm(h2, w2_ref[...])).astype(_BF)


def _sjk_deg_kernel(s_ref, v_ref, dinv_ref, mask_ref,
                    hn2_ref, hnT2_ref, xw1_ref,
                    jk_ref, dinv2_ref, mcat2_ref, s2_ref,
                    acc_ref, deg_ref, *, nj, bj):
    j = pl.program_id(1)

    @pl.when(j == 0)
    def _():
        acc_ref[...] = jnp.zeros_like(acc_ref)
        deg_ref[...] = jnp.zeros_like(deg_ref)

    acc_ref[...] += _dot(s_ref[...], v_ref[pl.ds(j * bj, bj), :])
    s2 = jnp.maximum(_dot(hn2_ref[...], hnT2_ref[:, pl.ds(j * bj, bj)]), 0.0)
    s2_ref[...] = s2.astype(_BF)
    deg_ref[...] += jnp.sum(s2, axis=1, keepdims=True)

    @pl.when(j == nj - 1)
    def _():
        jk_ref[...] = mask_ref[...] * dinv_ref[...] * acc_ref[...]
        dinv2 = _dinv_of(deg_ref[...])
        dinv2_ref[...] = dinv2
        mcat2_ref[...] = (dinv2 * xw1_ref[...].astype(jnp.float32)).astype(_BF)


def _sjk_kernel(s_ref, v_ref, dinv_ref, mask_ref, out_ref,
                acc_ref, *, nj, bj):
    j = pl.program_id(1)

    @pl.when(j == 0)
    def _():
        acc_ref[...] = jnp.zeros_like(acc_ref)

    acc_ref[...] += _dot(s_ref[...], v_ref[pl.ds(j * bj, bj), :])

    @pl.when(j == nj - 1)
    def _():
        out_ref[...] = mask_ref[...] * dinv_ref[...] * acc_ref[...]


def _full(shape):
    return pl.BlockSpec(shape, lambda i, j: (0, 0))


def _rowblk(b, w):
    return pl.BlockSpec((b, w), lambda i, j: (i, 0))


def kernel(X, initial_adj, minibatch_mask, W_ms1, W_ms2, W_init, W_gsl0, W_gsl1):
    n, d = X.shape
    f32 = jnp.float32
    maskf = minibatch_mask.astype(f32).reshape(n, 1)

    ni, nj = n // _BI, n // _BJ
    nia = n // _BIA
    params2 = pltpu.CompilerParams(
        dimension_semantics=("parallel", "arbitrary"))

    dinv0, a0bf = pl.pallas_call(
        functools.partial(_dinv0_kernel, nj=nia, bj=_BIA, d=d),
        grid=(nia, nia),
        in_specs=[pl.BlockSpec((_BIA, _BIA), lambda i, j: (i, j))],
        out_specs=[_rowblk(_BIA, 1),
                   pl.BlockSpec((_BIA, _BIA), lambda i, j: (i, j))],
        out_shape=[jax.ShapeDtypeStruct((n, 1), f32),
                   jax.ShapeDtypeStruct((n, n), _BF)],
        scratch_shapes=[pltpu.VMEM((_BIA, d), f32)],
        compiler_params=params2,
    )(initial_adj)

    xw1, m0 = pl.pallas_call(
        _xw_kernel,
        grid=(n // _BIA,),
        in_specs=[
            pl.BlockSpec((_BIA, d), lambda i: (i, 0)),
            pl.BlockSpec((d, d), lambda i: (0, 0)),
            pl.BlockSpec((d, d), lambda i: (0, 0)),
            pl.BlockSpec((_BIA, 1), lambda i: (i, 0)),
        ],
        out_specs=[pl.BlockSpec((_BIA, d), lambda i: (i, 0)),
                   pl.BlockSpec((_BIA, 2 * d), lambda i: (i, 0))],
        out_shape=[jax.ShapeDtypeStruct((n, d), _BF),
                   jax.ShapeDtypeStruct((n, 2 * d), _BF)],
        compiler_params=pltpu.CompilerParams(dimension_semantics=("parallel",)),
    )(X, W_ms1, W_init, dinv0)

    hn, hnT, g, gv0 = pl.pallas_call(
        functools.partial(_init_kernel, nj=nj, bj=_BJ, d=d),
        grid=(ni, nj),
        in_specs=[
            pl.BlockSpec((_BI, _BJ), lambda i, j: (i, j)),
            _full((n, 2 * d)),
            _rowblk(_BI, 1),
            _full((d, d)),
            _full((d, d)),
        ],
        out_specs=[
            _rowblk(_BI, d),
            pl.BlockSpec((d, _BI), lambda i, j: (0, i)),
            _rowblk(_BI, d),
            _rowblk(_BI, 2 * d),
        ],
        out_shape=[jax.ShapeDtypeStruct((n, d), _BF),
                   jax.ShapeDtypeStruct((d, n), _BF),
                   jax.ShapeDtypeStruct((n, d), _BF),
                   jax.ShapeDtypeStruct((n, 2 * d), _BF)],
        scratch_shapes=[pltpu.VMEM((_BI, 2 * d), f32)],
        compiler_params=params2,
    )(a0bf, m0, dinv0, W_ms2, W_gsl0)

    def deg_pass(hn_c, hnT_c, g_c, xw1_c):
        width = 2 * d
        return pl.pallas_call(
            functools.partial(_deg_kernel, nj=nj, bj=_BJ, d=d),
            grid=(ni, nj),
            in_specs=[
                _rowblk(_BI, d),
                _full((d, n)),
                _rowblk(_BI, d),
                _rowblk(_BI, d),
            ],
            out_specs=[_rowblk(_BI, 1), _rowblk(_BI, width),
                       pl.BlockSpec((_BI, _BJ), lambda i, j: (i, j))],
            out_shape=[jax.ShapeDtypeStruct((n, 1), f32),
                       jax.ShapeDtypeStruct((n, width), _BF),
                       jax.ShapeDtypeStruct((n, n), _BF)],
            scratch_shapes=[pltpu.VMEM((_BI, 1), f32)],
            compiler_params=params2,
        )(hn_c, hnT_c, g_c, xw1_c)

    def sjk_pass(s_c, v_c, dinv_c):
        return pl.pallas_call(
            functools.partial(_sjk_kernel, nj=nj, bj=_BJ),
            grid=(ni, nj),
            in_specs=[
                pl.BlockSpec((_BI, _BJ), lambda i, j: (i, j)),
                _full((n, d)),
                _rowblk(_BI, 1),
                _rowblk(_BI, 1),
            ],
            out_specs=_rowblk(_BI, d),
            out_shape=jax.ShapeDtypeStruct((n, d), f32),
            scratch_shapes=[pltpu.VMEM((_BI, d), f32)],
            compiler_params=params2,
        )(s_c, v_c, dinv_c, maskf)

    dinv1, mcat1, s1 = deg_pass(hn, hnT, g, xw1)

    hn2, hnT2, g2, v1, jk0 = pl.pallas_call(
        functools.partial(_fused_kernel, nj=nj, bj=_BJ, d=d),
        grid=(ni, nj),
        in_specs=[
            pl.BlockSpec((_BI, _BJ), lambda i, j: (i, j)),
            _full((n, 2 * d)),
            pl.BlockSpec((_BI, _BJ), lambda i, j: (i, j)),
            _full((n, 2 * d)),
            _rowblk(_BI, 1),
            _rowblk(_BI, 1),
            _full((d, d)),
            _full((d, d)),
            _rowblk(_BI, 1),
        ],
        out_specs=[
            _rowblk(_BI, d),
            pl.BlockSpec((d, _BI), lambda i, j: (0, i)),
            _rowblk(_BI, d),
            _rowblk(_BI, d),
            _rowblk(_BI, d),
        ],
        out_shape=[jax.ShapeDtypeStruct((n, d), _BF),
                   jax.ShapeDtypeStruct((d, n), _BF),
                   jax.ShapeDtypeStruct((n, d), _BF),
                   jax.ShapeDtypeStruct((n, d), _BF),
                   jax.ShapeDtypeStruct((n, d), f32)],
        scratch_shapes=[pltpu.VMEM((_BI, 2 * d), f32),
                        pltpu.VMEM((_BI, 2 * d), f32)],
        compiler_params=params2,
    )(s1, mcat1, a0bf, gv0, dinv1, dinv0, W_ms2, W_gsl1, maskf)

    jk1, dinv2, mcat2, s2 = pl.pallas_call(
        functools.partial(_sjk_deg_kernel, nj=nj, bj=_BJ),
        grid=(ni, nj),
        in_specs=[
            pl.BlockSpec((_BI, _BJ), lambda i, j: (i, j)),
            _full((n, d)),
            _rowblk(_BI, 1),
            _rowblk(_BI, 1),
            _rowblk(_BI, d),
            _full((d, n)),
            _rowblk(_BI, d),
        ],
        out_specs=[_rowblk(_BI, d), _rowblk(_BI, 1), _rowblk(_BI, d),
                   pl.BlockSpec((_BI, _BJ), lambda i, j: (i, j))],
        out_shape=[jax.ShapeDtypeStruct((n, d), f32),
                   jax.ShapeDtypeStruct((n, 1), f32),
                   jax.ShapeDtypeStruct((n, d), _BF),
                   jax.ShapeDtypeStruct((n, n), _BF)],
        scratch_shapes=[pltpu.VMEM((_BI, d), f32), pltpu.VMEM((_BI, 1), f32)],
        compiler_params=params2,
    )(s1, v1, dinv1, maskf, hn2, hnT2, xw1)

    v2 = pl.pallas_call(
        functools.partial(_fused2_kernel, nj=nj, bj=_BJ),
        grid=(ni, nj),
        in_specs=[
            pl.BlockSpec((_BI, _BJ), lambda i, j: (i, j)),
            _full((n, d)),
            _rowblk(_BI, 1),
            _full((d, d)),
        ],
        out_specs=_rowblk(_BI, d),
        out_shape=jax.ShapeDtypeStruct((n, d), _BF),
        scratch_shapes=[pltpu.VMEM((_BI, d), f32)],
        compiler_params=params2,
    )(s2, mcat2, dinv2, W_ms2)

    jk2 = sjk_pass(s2, v2, dinv2)

    return jnp.concatenate([jk0, jk1, jk2], axis=-1)


# speedup vs baseline: 1.0921x; 1.0921x over previous
import functools

import jax
import jax.numpy as jnp
from jax.experimental import pallas as pl
from jax.experimental.pallas import tpu as pltpu

_SKIP = 0.5

_BI = 512
_BJ = 1024
_BIA = 512

_BF = jnp.bfloat16


def _dot(a, b):
    return jax.lax.dot_general(
        a, b, (((1,), (0,)), ((), ())),
        preferred_element_type=jnp.float32)


def _dot_sm(a, b):
    return jax.lax.dot_general(
        a.astype(_BF), b.astype(_BF), (((1,), (0,)), ((), ())),
        preferred_element_type=jnp.float32)


def _dinv_of(deg):
    return jnp.where(deg > 0, 1.0 / jnp.sqrt(deg + 1e-12), 0.0)


def _dinv0_kernel(a_ref, x_ref, w1_ref, wi_ref,
                  dinv_ref, abf_ref, xw1_ref, m0_ref,
                  acc_ref, *, nj, bj, d):
    j = pl.program_id(1)

    @pl.when(j == 0)
    def _():
        acc_ref[...] = jnp.zeros_like(acc_ref)

    abf = a_ref[...].astype(_BF)
    abf_ref[...] = abf
    ones = jnp.ones((bj, d), dtype=_BF)
    acc_ref[...] += _dot(abf, ones)

    @pl.when(j == nj - 1)
    def _():
        d0 = _dinv_of(acc_ref[:, :1])
        dinv_ref[...] = d0
        xw1 = _dot_sm(x_ref[...], w1_ref[...])
        xwi = _dot_sm(x_ref[...], wi_ref[...])
        xw1_ref[...] = xw1.astype(_BF)
        m0_ref[:, :d] = (d0 * xw1).astype(_BF)
        m0_ref[:, d:] = (d0 * xwi).astype(_BF)


def _init_kernel(ah_ref, m0_ref, dinv0_ref, w2_ref, wg_ref,
                 hn_ref, hnT_ref, g_ref, gv0_ref,
                 acc_ref, *, nj, bj, d):
    j = pl.program_id(1)

    @pl.when(j == 0)
    def _():
        acc_ref[...] = jnp.zeros_like(acc_ref)

    acc_ref[...] += _dot(ah_ref[...], m0_ref[pl.ds(j * bj, bj), :])

    @pl.when(j == nj - 1)
    def _():
        d0 = dinv0_ref[...]
        p = acc_ref[...] * d0
        h_c0 = jnp.maximum(p[:, :d], 0.0)
        h = jnp.maximum(p[:, d:], 0.0)
        nrm = jnp.sqrt(jnp.sum(h * h, axis=1, keepdims=True)) + 1e-8
        hn = (h / nrm).astype(_BF)
        hn_ref[...] = hn
        hnT_ref[...] = hn.T
        g = _dot_sm(h, wg_ref[...])
        g_ref[...] = g.astype(_BF)
        gv0_ref[:, :d] = (d0 * g).astype(_BF)
        v0 = d0 * _dot_sm(h_c0, w2_ref[...])
        gv0_ref[:, d:] = v0.astype(_BF)


def _deg_kernel(hn_ref, hnT_ref, g_ref, xw1_ref, dinv_ref, mcat_ref,
                acc_ref, *, nj, bj, d):
    j = pl.program_id(1)

    @pl.when(j == 0)
    def _():
        acc_ref[...] = jnp.zeros_like(acc_ref)

    s = jnp.maximum(_dot(hn_ref[...], hnT_ref[:, pl.ds(j * bj, bj)]), 0.0)
    acc_ref[...] += jnp.sum(s, axis=1, keepdims=True)

    @pl.when(j == nj - 1)
    def _():
        dinv = _dinv_of(acc_ref[...])
        dinv_ref[...] = dinv
        mcat_ref[:, :d] = (dinv * g_ref[...].astype(jnp.float32)).astype(_BF)
        mcat_ref[:, d:] = (dinv * xw1_ref[...].astype(jnp.float32)).astype(_BF)


def _fused_kernel(hn_ref, hnT_ref, mcat_ref, ah_ref, gv0_ref, dinv_ref,
                  dinv0_ref, w2_ref, wgn_ref, mask_ref,
                  hn2_ref, hnT2_ref, g2_ref, v_ref, jk0_ref,
                  accs_ref, accc_ref, *, nj, bj, d):
    j = pl.program_id(1)

    @pl.when(j == 0)
    def _():
        accs_ref[...] = jnp.zeros_like(accs_ref)
        accc_ref[...] = jnp.zeros_like(accc_ref)

    s = jnp.maximum(_dot(hn_ref[...], hnT_ref[:, pl.ds(j * bj, bj)]), 0.0)
    accs_ref[...] += _dot(s.astype(_BF), mcat_ref[pl.ds(j * bj, bj), :])
    accc_ref[...] += _dot(ah_ref[...], gv0_ref[pl.ds(j * bj, bj), :])

    @pl.when(j == nj - 1)
    def _():
        dinv = dinv_ref[...]
        d0 = dinv0_ref[...]
        h_new = jnp.maximum(
            _SKIP * d0 * accc_ref[:, :d] + (1.0 - _SKIP) * dinv * accs_ref[:, :d],
            0.0)
        h2 = jnp.maximum(dinv * accs_ref[:, d:], 0.0)
        jk0_ref[...] = mask_ref[...] * d0 * accc_ref[:, d:]
        v_ref[...] = (dinv * _dot_sm(h2, w2_ref[...])).astype(_BF)
        nrm = jnp.sqrt(jnp.sum(h_new * h_new, axis=1, keepdims=True)) + 1e-8
        hn2 = (h_new / nrm).astype(_BF)
        hn2_ref[...] = hn2
        hnT2_ref[...] = hn2.T
        g2_ref[...] = _dot_sm(h_new, wgn_ref[...]).astype(_BF)


def _fused2_kernel(hn_ref, hnT_ref, m_ref, dinv_ref, w2_ref, v_ref,
                   acc_ref, *, nj, bj):
    j = pl.program_id(1)

    @pl.when(j == 0)
    def _():
        acc_ref[...] = jnp.zeros_like(acc_ref)

    s = jnp.maximum(_dot(hn_ref[...], hnT_ref[:, pl.ds(j * bj, bj)]), 0.0)
    acc_ref[...] += _dot(s.astype(_BF), m_ref[pl.ds(j * bj, bj), :])

    @pl.when(j == nj - 1)
    def _():
        dinv = dinv_ref[...]
        h2 = jnp.maximum(dinv * acc_ref[...], 0.0)
        v_ref[...] = (dinv * _dot_sm(h2, w2_ref[...])).astype(_BF)


def _sjk_deg_kernel(hn_ref, hnT_ref, v_ref, dinv_ref, mask_ref,
                    hn2_ref, hnT2_ref, xw1_ref,
                    jk_ref, dinv2_ref, mcat2_ref,
                    acc_ref, deg_ref, *, nj, bj):
    j = pl.program_id(1)

    @pl.when(j == 0)
    def _():
        acc_ref[...] = jnp.zeros_like(acc_ref)
        deg_ref[...] = jnp.zeros_like(deg_ref)

    s = jnp.maximum(_dot(hn_ref[...], hnT_ref[:, pl.ds(j * bj, bj)]), 0.0)
    acc_ref[...] += _dot(s.astype(_BF), v_ref[pl.ds(j * bj, bj), :])
    s2 = jnp.maximum(_dot(hn2_ref[...], hnT2_ref[:, pl.ds(j * bj, bj)]), 0.0)
    deg_ref[...] += jnp.sum(s2, axis=1, keepdims=True)

    @pl.when(j == nj - 1)
    def _():
        jk_ref[...] = mask_ref[...] * dinv_ref[...] * acc_ref[...]
        dinv2 = _dinv_of(deg_ref[...])
        dinv2_ref[...] = dinv2
        mcat2_ref[...] = (dinv2 * xw1_ref[...].astype(jnp.float32)).astype(_BF)


def _sjk_kernel(hn_ref, hnT_ref, v_ref, dinv_ref, mask_ref, out_ref,
                acc_ref, *, nj, bj):
    j = pl.program_id(1)

    @pl.when(j == 0)
    def _():
        acc_ref[...] = jnp.zeros_like(acc_ref)

    s = jnp.maximum(_dot(hn_ref[...], hnT_ref[:, pl.ds(j * bj, bj)]), 0.0)
    acc_ref[...] += _dot(s.astype(_BF), v_ref[pl.ds(j * bj, bj), :])

    @pl.when(j == nj - 1)
    def _():
        out_ref[...] = mask_ref[...] * dinv_ref[...] * acc_ref[...]


def _full(shape):
    return pl.BlockSpec(shape, lambda i, j: (0, 0))


def _rowblk(b, w):
    return pl.BlockSpec((b, w), lambda i, j: (i, 0))


def kernel(X, initial_adj, minibatch_mask, W_ms1, W_ms2, W_init, W_gsl0, W_gsl1):
    n, d = X.shape
    f32 = jnp.float32
    maskf = minibatch_mask.astype(f32).reshape(n, 1)

    ni, nj = n // _BI, n // _BJ
    nia = n // _BIA
    params2 = pltpu.CompilerParams(
        dimension_semantics=("parallel", "arbitrary"))

    dinv0, a0bf, xw1, m0 = pl.pallas_call(
        functools.partial(_dinv0_kernel, nj=nia, bj=_BIA, d=d),
        grid=(nia, nia),
        in_specs=[pl.BlockSpec((_BIA, _BIA), lambda i, j: (i, j)),
                  _rowblk(_BIA, d),
                  _full((d, d)),
                  _full((d, d))],
        out_specs=[_rowblk(_BIA, 1),
                   pl.BlockSpec((_BIA, _BIA), lambda i, j: (i, j)),
                   _rowblk(_BIA, d),
                   _rowblk(_BIA, 2 * d)],
        out_shape=[jax.ShapeDtypeStruct((n, 1), f32),
                   jax.ShapeDtypeStruct((n, n), _BF),
                   jax.ShapeDtypeStruct((n, d), _BF),
                   jax.ShapeDtypeStruct((n, 2 * d), _BF)],
        scratch_shapes=[pltpu.VMEM((_BIA, d), f32)],
        compiler_params=params2,
    )(initial_adj, X, W_ms1, W_init)

    hn, hnT, g, gv0 = pl.pallas_call(
        functools.partial(_init_kernel, nj=nj, bj=_BJ, d=d),
        grid=(ni, nj),
        in_specs=[
            pl.BlockSpec((_BI, _BJ), lambda i, j: (i, j)),
            _full((n, 2 * d)),
            _rowblk(_BI, 1),
            _full((d, d)),
            _full((d, d)),
        ],
        out_specs=[
            _rowblk(_BI, d),
            pl.BlockSpec((d, _BI), lambda i, j: (0, i)),
            _rowblk(_BI, d),
            _rowblk(_BI, 2 * d),
        ],
        out_shape=[jax.ShapeDtypeStruct((n, d), _BF),
                   jax.ShapeDtypeStruct((d, n), _BF),
                   jax.ShapeDtypeStruct((n, d), _BF),
                   jax.ShapeDtypeStruct((n, 2 * d), _BF)],
        scratch_shapes=[pltpu.VMEM((_BI, 2 * d), f32)],
        compiler_params=params2,
    )(a0bf, m0, dinv0, W_ms2, W_gsl0)

    def deg_pass(hn_c, hnT_c, g_c, xw1_c):
        width = 2 * d
        return pl.pallas_call(
            functools.partial(_deg_kernel, nj=nj, bj=_BJ, d=d),
            grid=(ni, nj),
            in_specs=[
                _rowblk(_BI, d),
                _full((d, n)),
                _rowblk(_BI, d),
                _rowblk(_BI, d),
            ],
            out_specs=[_rowblk(_BI, 1), _rowblk(_BI, width)],
            out_shape=[jax.ShapeDtypeStruct((n, 1), f32),
                       jax.ShapeDtypeStruct((n, width), _BF)],
            scratch_shapes=[pltpu.VMEM((_BI, 1), f32)],
            compiler_params=params2,
        )(hn_c, hnT_c, g_c, xw1_c)

    def sjk_pass(hn_c, hnT_c, v_c, dinv_c):
        return pl.pallas_call(
            functools.partial(_sjk_kernel, nj=nj, bj=_BJ),
            grid=(ni, nj),
            in_specs=[
                _rowblk(_BI, d),
                _full((d, n)),
                _full((n, d)),
                _rowblk(_BI, 1),
                _rowblk(_BI, 1),
            ],
            out_specs=_rowblk(_BI, d),
            out_shape=jax.ShapeDtypeStruct((n, d), f32),
            scratch_shapes=[pltpu.VMEM((_BI, d), f32)],
            compiler_params=params2,
        )(hn_c, hnT_c, v_c, dinv_c, maskf)

    dinv1, mcat1 = deg_pass(hn, hnT, g, xw1)

    hn2, hnT2, g2, v1, jk0 = pl.pallas_call(
        functools.partial(_fused_kernel, nj=nj, bj=_BJ, d=d),
        grid=(ni, nj),
        in_specs=[
            _rowblk(_BI, d),
            _full((d, n)),
            _full((n, 2 * d)),
            pl.BlockSpec((_BI, _BJ), lambda i, j: (i, j)),
            _full((n, 2 * d)),
            _rowblk(_BI, 1),
            _rowblk(_BI, 1),
            _full((d, d)),
            _full((d, d)),
            _rowblk(_BI, 1),
        ],
        out_specs=[
            _rowblk(_BI, d),
            pl.BlockSpec((d, _BI), lambda i, j: (0, i)),
            _rowblk(_BI, d),
            _rowblk(_BI, d),
            _rowblk(_BI, d),
        ],
        out_shape=[jax.ShapeDtypeStruct((n, d), _BF),
                   jax.ShapeDtypeStruct((d, n), _BF),
                   jax.ShapeDtypeStruct((n, d), _BF),
                   jax.ShapeDtypeStruct((n, d), _BF),
                   jax.ShapeDtypeStruct((n, d), f32)],
        scratch_shapes=[pltpu.VMEM((_BI, 2 * d), f32),
                        pltpu.VMEM((_BI, 2 * d), f32)],
        compiler_params=params2,
    )(hn, hnT, mcat1, a0bf, gv0, dinv1, dinv0, W_ms2, W_gsl1, maskf)

    jk1, dinv2, mcat2 = pl.pallas_call(
        functools.partial(_sjk_deg_kernel, nj=nj, bj=_BJ),
        grid=(ni, nj),
        in_specs=[
            _rowblk(_BI, d),
            _full((d, n)),
            _full((n, d)),
            _rowblk(_BI, 1),
            _rowblk(_BI, 1),
            _rowblk(_BI, d),
            _full((d, n)),
            _rowblk(_BI, d),
        ],
        out_specs=[_rowblk(_BI, d), _rowblk(_BI, 1), _rowblk(_BI, d)],
        out_shape=[jax.ShapeDtypeStruct((n, d), f32),
                   jax.ShapeDtypeStruct((n, 1), f32),
                   jax.ShapeDtypeStruct((n, d), _BF)],
        scratch_shapes=[pltpu.VMEM((_BI, d), f32), pltpu.VMEM((_BI, 1), f32)],
        compiler_params=params2,
    )(hn, hnT, v1, dinv1, maskf, hn2, hnT2, xw1)

    v2 = pl.pallas_call(
        functools.partial(_fused2_kernel, nj=nj, bj=_BJ),
        grid=(ni, nj),
        in_specs=[
            _rowblk(_BI, d),
            _full((d, n)),
            _full((n, d)),
            _rowblk(_BI, 1),
            _full((d, d)),
        ],
        out_specs=_rowblk(_BI, d),
        out_shape=jax.ShapeDtypeStruct((n, d), _BF),
        scratch_shapes=[pltpu.VMEM((_BI, d), f32)],
        compiler_params=params2,
    )(hn2, hnT2, mcat2, dinv2, W_ms2)

    jk2 = sjk_pass(hn2, hnT2, v2, dinv2)

    return jnp.concatenate([jk0, jk1, jk2], axis=-1)


# speedup vs baseline: 1.2788x; 1.1709x over previous
import functools

import jax
import jax.numpy as jnp
from jax.experimental import pallas as pl
from jax.experimental.pallas import tpu as pltpu

_SKIP = 0.5

_BI = 1024
_BJ = 1024
_BIA = 512

_BF = jnp.bfloat16


def _dot(a, b):
    return jax.lax.dot_general(
        a, b, (((1,), (0,)), ((), ())),
        preferred_element_type=jnp.float32)


def _dot_sm(a, b):
    return jax.lax.dot_general(
        a.astype(_BF), b.astype(_BF), (((1,), (0,)), ((), ())),
        preferred_element_type=jnp.float32)


def _dinv_of(deg):
    return jnp.where(deg > 0, 1.0 / jnp.sqrt(deg + 1e-12), 0.0)


def _dinv0_kernel(a_ref, x_ref, w1_ref, wi_ref,
                  dinv_ref, abf_ref, xw1_ref, m0_ref,
                  acc_ref, *, nj, bj, d):
    j = pl.program_id(1)

    @pl.when(j == 0)
    def _():
        acc_ref[...] = jnp.zeros_like(acc_ref)

    abf = a_ref[...].astype(_BF)
    abf_ref[...] = abf
    ones = jnp.ones((bj, d), dtype=_BF)
    acc_ref[...] += _dot(abf, ones)

    @pl.when(j == nj - 1)
    def _():
        d0 = _dinv_of(acc_ref[:, :1])
        dinv_ref[...] = d0
        xw1 = _dot_sm(x_ref[...], w1_ref[...])
        xwi = _dot_sm(x_ref[...], wi_ref[...])
        xw1_ref[...] = xw1.astype(_BF)
        m0_ref[:, :d] = (d0 * xw1).astype(_BF)
        m0_ref[:, d:] = (d0 * xwi).astype(_BF)


def _init_kernel(ah_ref, m0_ref, dinv0_ref, w2_ref, wg_ref,
                 hn_ref, hnT_ref, g_ref, gv0_ref,
                 acc_ref, *, nj, bj, d):
    j = pl.program_id(1)

    @pl.when(j == 0)
    def _():
        acc_ref[...] = jnp.zeros_like(acc_ref)

    acc_ref[...] += _dot(ah_ref[...], m0_ref[pl.ds(j * bj, bj), :])

    @pl.when(j == nj - 1)
    def _():
        d0 = dinv0_ref[...]
        p = acc_ref[...] * d0
        h_c0 = jnp.maximum(p[:, :d], 0.0)
        h = jnp.maximum(p[:, d:], 0.0)
        nrm = jnp.sqrt(jnp.sum(h * h, axis=1, keepdims=True)) + 1e-8
        hn = (h / nrm).astype(_BF)
        hn_ref[...] = hn
        hnT_ref[...] = hn.T
        g = _dot_sm(h, wg_ref[...])
        g_ref[...] = g.astype(_BF)
        gv0_ref[:, :d] = (d0 * g).astype(_BF)
        v0 = d0 * _dot_sm(h_c0, w2_ref[...])
        gv0_ref[:, d:] = v0.astype(_BF)


def _deg_kernel(hn_ref, hnT_ref, g_ref, xw1_ref, dinv_ref, mcat_ref,
                acc_ref, *, nj, bj, d):
    j = pl.program_id(1)

    @pl.when(j == 0)
    def _():
        acc_ref[...] = jnp.zeros_like(acc_ref)

    s = jnp.maximum(_dot(hn_ref[...], hnT_ref[:, pl.ds(j * bj, bj)]), 0.0)
    acc_ref[...] += jnp.sum(s, axis=1, keepdims=True)

    @pl.when(j == nj - 1)
    def _():
        dinv = _dinv_of(acc_ref[...])
        dinv_ref[...] = dinv
        mcat_ref[:, :d] = (dinv * g_ref[...].astype(jnp.float32)).astype(_BF)
        mcat_ref[:, d:] = (dinv * xw1_ref[...].astype(jnp.float32)).astype(_BF)


def _fused_kernel(hn_ref, hnT_ref, mcat_ref, ah_ref, gv0_ref, dinv_ref,
                  dinv0_ref, w2_ref, wgn_ref, mask_ref,
                  hn2_ref, hnT2_ref, g2_ref, v_ref, jk0_ref,
                  accs_ref, accc_ref, *, nj, bj, d):
    j = pl.program_id(1)

    @pl.when(j == 0)
    def _():
        accs_ref[...] = jnp.zeros_like(accs_ref)
        accc_ref[...] = jnp.zeros_like(accc_ref)

    s = jnp.maximum(_dot(hn_ref[...], hnT_ref[:, pl.ds(j * bj, bj)]), 0.0)
    accs_ref[...] += _dot(s.astype(_BF), mcat_ref[pl.ds(j * bj, bj), :])
    accc_ref[...] += _dot(ah_ref[...], gv0_ref[pl.ds(j * bj, bj), :])

    @pl.when(j == nj - 1)
    def _():
        dinv = dinv_ref[...]
        d0 = dinv0_ref[...]
        h_new = jnp.maximum(
            _SKIP * d0 * accc_ref[:, :d] + (1.0 - _SKIP) * dinv * accs_ref[:, :d],
            0.0)
        h2 = jnp.maximum(dinv * accs_ref[:, d:], 0.0)
        jk0_ref[...] = mask_ref[...] * d0 * accc_ref[:, d:]
        v_ref[...] = (dinv * _dot_sm(h2, w2_ref[...])).astype(_BF)
        nrm = jnp.sqrt(jnp.sum(h_new * h_new, axis=1, keepdims=True)) + 1e-8
        hn2 = (h_new / nrm).astype(_BF)
        hn2_ref[...] = hn2
        hnT2_ref[...] = hn2.T
        g2_ref[...] = _dot_sm(h_new, wgn_ref[...]).astype(_BF)


def _fused2_kernel(hn_ref, hnT_ref, m_ref, dinv_ref, w2_ref, v_ref,
                   acc_ref, *, nj, bj):
    j = pl.program_id(1)

    @pl.when(j == 0)
    def _():
        acc_ref[...] = jnp.zeros_like(acc_ref)

    s = jnp.maximum(_dot(hn_ref[...], hnT_ref[:, pl.ds(j * bj, bj)]), 0.0)
    acc_ref[...] += _dot(s.astype(_BF), m_ref[pl.ds(j * bj, bj), :])

    @pl.when(j == nj - 1)
    def _():
        dinv = dinv_ref[...]
        h2 = jnp.maximum(dinv * acc_ref[...], 0.0)
        v_ref[...] = (dinv * _dot_sm(h2, w2_ref[...])).astype(_BF)


def _sjk_deg_kernel(hn_ref, hnT_ref, v_ref, dinv_ref, mask_ref,
                    hn2_ref, hnT2_ref, xw1_ref,
                    jk_ref, dinv2_ref, mcat2_ref,
                    acc_ref, deg_ref, *, nj, bj):
    j = pl.program_id(1)

    @pl.when(j == 0)
    def _():
        acc_ref[...] = jnp.zeros_like(acc_ref)
        deg_ref[...] = jnp.zeros_like(deg_ref)

    s = jnp.maximum(_dot(hn_ref[...], hnT_ref[:, pl.ds(j * bj, bj)]), 0.0)
    acc_ref[...] += _dot(s.astype(_BF), v_ref[pl.ds(j * bj, bj), :])
    s2 = jnp.maximum(_dot(hn2_ref[...], hnT2_ref[:, pl.ds(j * bj, bj)]), 0.0)
    deg_ref[...] += jnp.sum(s2, axis=1, keepdims=True)

    @pl.when(j == nj - 1)
    def _():
        jk_ref[...] = mask_ref[...] * dinv_ref[...] * acc_ref[...]
        dinv2 = _dinv_of(deg_ref[...])
        dinv2_ref[...] = dinv2
        mcat2_ref[...] = (dinv2 * xw1_ref[...].astype(jnp.float32)).astype(_BF)


def _sjk_kernel(hn_ref, hnT_ref, v_ref, dinv_ref, mask_ref, out_ref,
                acc_ref, *, nj, bj):
    j = pl.program_id(1)

    @pl.when(j == 0)
    def _():
        acc_ref[...] = jnp.zeros_like(acc_ref)

    s = jnp.maximum(_dot(hn_ref[...], hnT_ref[:, pl.ds(j * bj, bj)]), 0.0)
    acc_ref[...] += _dot(s.astype(_BF), v_ref[pl.ds(j * bj, bj), :])

    @pl.when(j == nj - 1)
    def _():
        out_ref[...] = mask_ref[...] * dinv_ref[...] * acc_ref[...]


def _full(shape):
    return pl.BlockSpec(shape, lambda i, j: (0, 0))


def _rowblk(b, w):
    return pl.BlockSpec((b, w), lambda i, j: (i, 0))


def kernel(X, initial_adj, minibatch_mask, W_ms1, W_ms2, W_init, W_gsl0, W_gsl1):
    n, d = X.shape
    f32 = jnp.float32
    maskf = minibatch_mask.astype(f32).reshape(n, 1)

    ni, nj = n // _BI, n // _BJ
    nia = n // _BIA
    params2 = pltpu.CompilerParams(
        dimension_semantics=("parallel", "arbitrary"))

    dinv0, a0bf, xw1, m0 = pl.pallas_call(
        functools.partial(_dinv0_kernel, nj=nia, bj=_BIA, d=d),
        grid=(nia, nia),
        in_specs=[pl.BlockSpec((_BIA, _BIA), lambda i, j: (i, j)),
                  _rowblk(_BIA, d),
                  _full((d, d)),
                  _full((d, d))],
        out_specs=[_rowblk(_BIA, 1),
                   pl.BlockSpec((_BIA, _BIA), lambda i, j: (i, j)),
                   _rowblk(_BIA, d),
                   _rowblk(_BIA, 2 * d)],
        out_shape=[jax.ShapeDtypeStruct((n, 1), f32),
                   jax.ShapeDtypeStruct((n, n), _BF),
                   jax.ShapeDtypeStruct((n, d), _BF),
                   jax.ShapeDtypeStruct((n, 2 * d), _BF)],
        scratch_shapes=[pltpu.VMEM((_BIA, d), f32)],
        compiler_params=params2,
    )(initial_adj, X, W_ms1, W_init)

    hn, hnT, g, gv0 = pl.pallas_call(
        functools.partial(_init_kernel, nj=nj, bj=_BJ, d=d),
        grid=(ni, nj),
        in_specs=[
            pl.BlockSpec((_BI, _BJ), lambda i, j: (i, j)),
            _full((n, 2 * d)),
            _rowblk(_BI, 1),
            _full((d, d)),
            _full((d, d)),
        ],
        out_specs=[
            _rowblk(_BI, d),
            pl.BlockSpec((d, _BI), lambda i, j: (0, i)),
            _rowblk(_BI, d),
            _rowblk(_BI, 2 * d),
        ],
        out_shape=[jax.ShapeDtypeStruct((n, d), _BF),
                   jax.ShapeDtypeStruct((d, n), _BF),
                   jax.ShapeDtypeStruct((n, d), _BF),
                   jax.ShapeDtypeStruct((n, 2 * d), _BF)],
        scratch_shapes=[pltpu.VMEM((_BI, 2 * d), f32)],
        compiler_params=params2,
    )(a0bf, m0, dinv0, W_ms2, W_gsl0)

    def deg_pass(hn_c, hnT_c, g_c, xw1_c):
        width = 2 * d
        return pl.pallas_call(
            functools.partial(_deg_kernel, nj=nj, bj=_BJ, d=d),
            grid=(ni, nj),
            in_specs=[
                _rowblk(_BI, d),
                _full((d, n)),
                _rowblk(_BI, d),
                _rowblk(_BI, d),
            ],
            out_specs=[_rowblk(_BI, 1), _rowblk(_BI, width)],
            out_shape=[jax.ShapeDtypeStruct((n, 1), f32),
                       jax.ShapeDtypeStruct((n, width), _BF)],
            scratch_shapes=[pltpu.VMEM((_BI, 1), f32)],
            compiler_params=params2,
        )(hn_c, hnT_c, g_c, xw1_c)

    def sjk_pass(hn_c, hnT_c, v_c, dinv_c):
        return pl.pallas_call(
            functools.partial(_sjk_kernel, nj=nj, bj=_BJ),
            grid=(ni, nj),
            in_specs=[
                _rowblk(_BI, d),
                _full((d, n)),
                _full((n, d)),
                _rowblk(_BI, 1),
                _rowblk(_BI, 1),
            ],
            out_specs=_rowblk(_BI, d),
            out_shape=jax.ShapeDtypeStruct((n, d), f32),
            scratch_shapes=[pltpu.VMEM((_BI, d), f32)],
            compiler_params=params2,
        )(hn_c, hnT_c, v_c, dinv_c, maskf)

    dinv1, mcat1 = deg_pass(hn, hnT, g, xw1)

    hn2, hnT2, g2, v1, jk0 = pl.pallas_call(
        functools.partial(_fused_kernel, nj=nj, bj=_BJ, d=d),
        grid=(ni, nj),
        in_specs=[
            _rowblk(_BI, d),
            _full((d, n)),
            _full((n, 2 * d)),
            pl.BlockSpec((_BI, _BJ), lambda i, j: (i, j)),
            _full((n, 2 * d)),
            _rowblk(_BI, 1),
            _rowblk(_BI, 1),
            _full((d, d)),
            _full((d, d)),
            _rowblk(_BI, 1),
        ],
        out_specs=[
            _rowblk(_BI, d),
            pl.BlockSpec((d, _BI), lambda i, j: (0, i)),
            _rowblk(_BI, d),
            _rowblk(_BI, d),
            _rowblk(_BI, d),
        ],
        out_shape=[jax.ShapeDtypeStruct((n, d), _BF),
                   jax.ShapeDtypeStruct((d, n), _BF),
                   jax.ShapeDtypeStruct((n, d), _BF),
                   jax.ShapeDtypeStruct((n, d), _BF),
                   jax.ShapeDtypeStruct((n, d), f32)],
        scratch_shapes=[pltpu.VMEM((_BI, 2 * d), f32),
                        pltpu.VMEM((_BI, 2 * d), f32)],
        compiler_params=params2,
    )(hn, hnT, mcat1, a0bf, gv0, dinv1, dinv0, W_ms2, W_gsl1, maskf)

    jk1, dinv2, mcat2 = pl.pallas_call(
        functools.partial(_sjk_deg_kernel, nj=nj, bj=_BJ),
        grid=(ni, nj),
        in_specs=[
            _rowblk(_BI, d),
            _full((d, n)),
            _full((n, d)),
            _rowblk(_BI, 1),
            _rowblk(_BI, 1),
            _rowblk(_BI, d),
            _full((d, n)),
            _rowblk(_BI, d),
        ],
        out_specs=[_rowblk(_BI, d), _rowblk(_BI, 1), _rowblk(_BI, d)],
        out_shape=[jax.ShapeDtypeStruct((n, d), f32),
                   jax.ShapeDtypeStruct((n, 1), f32),
                   jax.ShapeDtypeStruct((n, d), _BF)],
        scratch_shapes=[pltpu.VMEM((_BI, d), f32), pltpu.VMEM((_BI, 1), f32)],
        compiler_params=params2,
    )(hn, hnT, v1, dinv1, maskf, hn2, hnT2, xw1)

    v2 = pl.pallas_call(
        functools.partial(_fused2_kernel, nj=nj, bj=_BJ),
        grid=(ni, nj),
        in_specs=[
            _rowblk(_BI, d),
            _full((d, n)),
            _full((n, d)),
            _rowblk(_BI, 1),
            _full((d, d)),
        ],
        out_specs=_rowblk(_BI, d),
        out_shape=jax.ShapeDtypeStruct((n, d), _BF),
        scratch_shapes=[pltpu.VMEM((_BI, d), f32)],
        compiler_params=params2,
    )(hn2, hnT2, mcat2, dinv2, W_ms2)

    jk2 = sjk_pass(hn2, hnT2, v2, dinv2)

    return jnp.concatenate([jk0, jk1, jk2], axis=-1)


# speedup vs baseline: 1.3331x; 1.0425x over previous
import functools

import jax
import jax.numpy as jnp
from jax.experimental import pallas as pl
from jax.experimental.pallas import tpu as pltpu

_SKIP = 0.5

_BI = 1024
_BJ = 2048
_BIA = 512

_BF = jnp.bfloat16


def _dot(a, b):
    return jax.lax.dot_general(
        a, b, (((1,), (0,)), ((), ())),
        preferred_element_type=jnp.float32)


def _dot_sm(a, b):
    return jax.lax.dot_general(
        a.astype(_BF), b.astype(_BF), (((1,), (0,)), ((), ())),
        preferred_element_type=jnp.float32)


def _dinv_of(deg):
    return jnp.where(deg > 0, 1.0 / jnp.sqrt(deg + 1e-12), 0.0)


def _dinv0_kernel(a_ref, x_ref, w1_ref, wi_ref,
                  dinv_ref, abf_ref, xw1_ref, m0_ref,
                  acc_ref, *, nj, bj, d):
    j = pl.program_id(1)

    @pl.when(j == 0)
    def _():
        acc_ref[...] = jnp.zeros_like(acc_ref)

    abf = a_ref[...].astype(_BF)
    abf_ref[...] = abf
    ones = jnp.ones((bj, d), dtype=_BF)
    acc_ref[...] += _dot(abf, ones)

    @pl.when(j == nj - 1)
    def _():
        d0 = _dinv_of(acc_ref[:, :1])
        dinv_ref[...] = d0
        xw1 = _dot_sm(x_ref[...], w1_ref[...])
        xwi = _dot_sm(x_ref[...], wi_ref[...])
        xw1_ref[...] = xw1.astype(_BF)
        m0_ref[:, :d] = (d0 * xw1).astype(_BF)
        m0_ref[:, d:] = (d0 * xwi).astype(_BF)


def _init_kernel(ah_ref, m0_ref, dinv0_ref, w2_ref, wg_ref,
                 hn_ref, hnT_ref, g_ref, gv0_ref,
                 acc_ref, *, nj, bj, d):
    j = pl.program_id(1)

    @pl.when(j == 0)
    def _():
        acc_ref[...] = jnp.zeros_like(acc_ref)

    acc_ref[...] += _dot(ah_ref[...], m0_ref[pl.ds(j * bj, bj), :])

    @pl.when(j == nj - 1)
    def _():
        d0 = dinv0_ref[...]
        p = acc_ref[...] * d0
        h_c0 = jnp.maximum(p[:, :d], 0.0)
        h = jnp.maximum(p[:, d:], 0.0)
        nrm = jnp.sqrt(jnp.sum(h * h, axis=1, keepdims=True)) + 1e-8
        hn = (h / nrm).astype(_BF)
        hn_ref[...] = hn
        hnT_ref[...] = hn.T
        g = _dot_sm(h, wg_ref[...])
        g_ref[...] = g.astype(_BF)
        gv0_ref[:, :d] = (d0 * g).astype(_BF)
        v0 = d0 * _dot_sm(h_c0, w2_ref[...])
        gv0_ref[:, d:] = v0.astype(_BF)


def _deg_kernel(hn_ref, hnT_ref, g_ref, xw1_ref, dinv_ref, mcat_ref,
                acc_ref, *, nj, bj, d):
    j = pl.program_id(1)

    @pl.when(j == 0)
    def _():
        acc_ref[...] = jnp.zeros_like(acc_ref)

    s = jnp.maximum(_dot(hn_ref[...], hnT_ref[:, pl.ds(j * bj, bj)]), 0.0)
    acc_ref[...] += jnp.sum(s, axis=1, keepdims=True)

    @pl.when(j == nj - 1)
    def _():
        dinv = _dinv_of(acc_ref[...])
        dinv_ref[...] = dinv
        mcat_ref[:, :d] = (dinv * g_ref[...].astype(jnp.float32)).astype(_BF)
        mcat_ref[:, d:] = (dinv * xw1_ref[...].astype(jnp.float32)).astype(_BF)


def _fused_kernel(hn_ref, hnT_ref, mcat_ref, ah_ref, gv0_ref, dinv_ref,
                  dinv0_ref, w2_ref, wgn_ref, mask_ref,
                  hn2_ref, hnT2_ref, g2_ref, v_ref, jk0_ref,
                  accs_ref, accc_ref, *, nj, bj, d):
    j = pl.program_id(1)

    @pl.when(j == 0)
    def _():
        accs_ref[...] = jnp.zeros_like(accs_ref)
        accc_ref[...] = jnp.zeros_like(accc_ref)

    s = jnp.maximum(_dot(hn_ref[...], hnT_ref[:, pl.ds(j * bj, bj)]), 0.0)
    accs_ref[...] += _dot(s.astype(_BF), mcat_ref[pl.ds(j * bj, bj), :])
    accc_ref[...] += _dot(ah_ref[...], gv0_ref[pl.ds(j * bj, bj), :])

    @pl.when(j == nj - 1)
    def _():
        dinv = dinv_ref[...]
        d0 = dinv0_ref[...]
        h_new = jnp.maximum(
            _SKIP * d0 * accc_ref[:, :d] + (1.0 - _SKIP) * dinv * accs_ref[:, :d],
            0.0)
        h2 = jnp.maximum(dinv * accs_ref[:, d:], 0.0)
        jk0_ref[...] = mask_ref[...] * d0 * accc_ref[:, d:]
        v_ref[...] = (dinv * _dot_sm(h2, w2_ref[...])).astype(_BF)
        nrm = jnp.sqrt(jnp.sum(h_new * h_new, axis=1, keepdims=True)) + 1e-8
        hn2 = (h_new / nrm).astype(_BF)
        hn2_ref[...] = hn2
        hnT2_ref[...] = hn2.T
        g2_ref[...] = _dot_sm(h_new, wgn_ref[...]).astype(_BF)


def _fused2_kernel(hn_ref, hnT_ref, m_ref, dinv_ref, w2_ref, v_ref,
                   acc_ref, *, nj, bj):
    j = pl.program_id(1)

    @pl.when(j == 0)
    def _():
        acc_ref[...] = jnp.zeros_like(acc_ref)

    s = jnp.maximum(_dot(hn_ref[...], hnT_ref[:, pl.ds(j * bj, bj)]), 0.0)
    acc_ref[...] += _dot(s.astype(_BF), m_ref[pl.ds(j * bj, bj), :])

    @pl.when(j == nj - 1)
    def _():
        dinv = dinv_ref[...]
        h2 = jnp.maximum(dinv * acc_ref[...], 0.0)
        v_ref[...] = (dinv * _dot_sm(h2, w2_ref[...])).astype(_BF)


def _sjk_deg_kernel(hn_ref, hnT_ref, v_ref, dinv_ref, mask_ref,
                    hn2_ref, hnT2_ref, xw1_ref,
                    jk_ref, dinv2_ref, mcat2_ref,
                    acc_ref, deg_ref, *, nj, bj):
    j = pl.program_id(1)

    @pl.when(j == 0)
    def _():
        acc_ref[...] = jnp.zeros_like(acc_ref)
        deg_ref[...] = jnp.zeros_like(deg_ref)

    s = jnp.maximum(_dot(hn_ref[...], hnT_ref[:, pl.ds(j * bj, bj)]), 0.0)
    acc_ref[...] += _dot(s.astype(_BF), v_ref[pl.ds(j * bj, bj), :])
    s2 = jnp.maximum(_dot(hn2_ref[...], hnT2_ref[:, pl.ds(j * bj, bj)]), 0.0)
    deg_ref[...] += jnp.sum(s2, axis=1, keepdims=True)

    @pl.when(j == nj - 1)
    def _():
        jk_ref[...] = mask_ref[...] * dinv_ref[...] * acc_ref[...]
        dinv2 = _dinv_of(deg_ref[...])
        dinv2_ref[...] = dinv2
        mcat2_ref[...] = (dinv2 * xw1_ref[...].astype(jnp.float32)).astype(_BF)


def _sjk_kernel(hn_ref, hnT_ref, v_ref, dinv_ref, mask_ref, out_ref,
                acc_ref, *, nj, bj):
    j = pl.program_id(1)

    @pl.when(j == 0)
    def _():
        acc_ref[...] = jnp.zeros_like(acc_ref)

    s = jnp.maximum(_dot(hn_ref[...], hnT_ref[:, pl.ds(j * bj, bj)]), 0.0)
    acc_ref[...] += _dot(s.astype(_BF), v_ref[pl.ds(j * bj, bj), :])

    @pl.when(j == nj - 1)
    def _():
        out_ref[...] = mask_ref[...] * dinv_ref[...] * acc_ref[...]


def _full(shape):
    return pl.BlockSpec(shape, lambda i, j: (0, 0))


def _rowblk(b, w):
    return pl.BlockSpec((b, w), lambda i, j: (i, 0))


def kernel(X, initial_adj, minibatch_mask, W_ms1, W_ms2, W_init, W_gsl0, W_gsl1):
    n, d = X.shape
    f32 = jnp.float32
    maskf = minibatch_mask.astype(f32).reshape(n, 1)

    ni, nj = n // _BI, n // _BJ
    nia = n // _BIA
    params2 = pltpu.CompilerParams(
        dimension_semantics=("parallel", "arbitrary"))

    dinv0, a0bf, xw1, m0 = pl.pallas_call(
        functools.partial(_dinv0_kernel, nj=nia, bj=_BIA, d=d),
        grid=(nia, nia),
        in_specs=[pl.BlockSpec((_BIA, _BIA), lambda i, j: (i, j)),
                  _rowblk(_BIA, d),
                  _full((d, d)),
                  _full((d, d))],
        out_specs=[_rowblk(_BIA, 1),
                   pl.BlockSpec((_BIA, _BIA), lambda i, j: (i, j)),
                   _rowblk(_BIA, d),
                   _rowblk(_BIA, 2 * d)],
        out_shape=[jax.ShapeDtypeStruct((n, 1), f32),
                   jax.ShapeDtypeStruct((n, n), _BF),
                   jax.ShapeDtypeStruct((n, d), _BF),
                   jax.ShapeDtypeStruct((n, 2 * d), _BF)],
        scratch_shapes=[pltpu.VMEM((_BIA, d), f32)],
        compiler_params=params2,
    )(initial_adj, X, W_ms1, W_init)

    hn, hnT, g, gv0 = pl.pallas_call(
        functools.partial(_init_kernel, nj=nj, bj=_BJ, d=d),
        grid=(ni, nj),
        in_specs=[
            pl.BlockSpec((_BI, _BJ), lambda i, j: (i, j)),
            _full((n, 2 * d)),
            _rowblk(_BI, 1),
            _full((d, d)),
            _full((d, d)),
        ],
        out_specs=[
            _rowblk(_BI, d),
            pl.BlockSpec((d, _BI), lambda i, j: (0, i)),
            _rowblk(_BI, d),
            _rowblk(_BI, 2 * d),
        ],
        out_shape=[jax.ShapeDtypeStruct((n, d), _BF),
                   jax.ShapeDtypeStruct((d, n), _BF),
                   jax.ShapeDtypeStruct((n, d), _BF),
                   jax.ShapeDtypeStruct((n, 2 * d), _BF)],
        scratch_shapes=[pltpu.VMEM((_BI, 2 * d), f32)],
        compiler_params=params2,
    )(a0bf, m0, dinv0, W_ms2, W_gsl0)

    def deg_pass(hn_c, hnT_c, g_c, xw1_c):
        width = 2 * d
        return pl.pallas_call(
            functools.partial(_deg_kernel, nj=nj, bj=_BJ, d=d),
            grid=(ni, nj),
            in_specs=[
                _rowblk(_BI, d),
                _full((d, n)),
                _rowblk(_BI, d),
                _rowblk(_BI, d),
            ],
            out_specs=[_rowblk(_BI, 1), _rowblk(_BI, width)],
            out_shape=[jax.ShapeDtypeStruct((n, 1), f32),
                       jax.ShapeDtypeStruct((n, width), _BF)],
            scratch_shapes=[pltpu.VMEM((_BI, 1), f32)],
            compiler_params=params2,
        )(hn_c, hnT_c, g_c, xw1_c)

    def sjk_pass(hn_c, hnT_c, v_c, dinv_c):
        return pl.pallas_call(
            functools.partial(_sjk_kernel, nj=nj, bj=_BJ),
            grid=(ni, nj),
            in_specs=[
                _rowblk(_BI, d),
                _full((d, n)),
                _full((n, d)),
                _rowblk(_BI, 1),
                _rowblk(_BI, 1),
            ],
            out_specs=_rowblk(_BI, d),
            out_shape=jax.ShapeDtypeStruct((n, d), f32),
            scratch_shapes=[pltpu.VMEM((_BI, d), f32)],
            compiler_params=params2,
        )(hn_c, hnT_c, v_c, dinv_c, maskf)

    dinv1, mcat1 = deg_pass(hn, hnT, g, xw1)

    hn2, hnT2, g2, v1, jk0 = pl.pallas_call(
        functools.partial(_fused_kernel, nj=nj, bj=_BJ, d=d),
        grid=(ni, nj),
        in_specs=[
            _rowblk(_BI, d),
            _full((d, n)),
            _full((n, 2 * d)),
            pl.BlockSpec((_BI, _BJ), lambda i, j: (i, j)),
            _full((n, 2 * d)),
            _rowblk(_BI, 1),
            _rowblk(_BI, 1),
            _full((d, d)),
            _full((d, d)),
            _rowblk(_BI, 1),
        ],
        out_specs=[
            _rowblk(_BI, d),
            pl.BlockSpec((d, _BI), lambda i, j: (0, i)),
            _rowblk(_BI, d),
            _rowblk(_BI, d),
            _rowblk(_BI, d),
        ],
        out_shape=[jax.ShapeDtypeStruct((n, d), _BF),
                   jax.ShapeDtypeStruct((d, n), _BF),
                   jax.ShapeDtypeStruct((n, d), _BF),
                   jax.ShapeDtypeStruct((n, d), _BF),
                   jax.ShapeDtypeStruct((n, d), f32)],
        scratch_shapes=[pltpu.VMEM((_BI, 2 * d), f32),
                        pltpu.VMEM((_BI, 2 * d), f32)],
        compiler_params=params2,
    )(hn, hnT, mcat1, a0bf, gv0, dinv1, dinv0, W_ms2, W_gsl1, maskf)

    jk1, dinv2, mcat2 = pl.pallas_call(
        functools.partial(_sjk_deg_kernel, nj=nj, bj=_BJ),
        grid=(ni, nj),
        in_specs=[
            _rowblk(_BI, d),
            _full((d, n)),
            _full((n, d)),
            _rowblk(_BI, 1),
            _rowblk(_BI, 1),
            _rowblk(_BI, d),
            _full((d, n)),
            _rowblk(_BI, d),
        ],
        out_specs=[_rowblk(_BI, d), _rowblk(_BI, 1), _rowblk(_BI, d)],
        out_shape=[jax.ShapeDtypeStruct((n, d), f32),
                   jax.ShapeDtypeStruct((n, 1), f32),
                   jax.ShapeDtypeStruct((n, d), _BF)],
        scratch_shapes=[pltpu.VMEM((_BI, d), f32), pltpu.VMEM((_BI, 1), f32)],
        compiler_params=params2,
    )(hn, hnT, v1, dinv1, maskf, hn2, hnT2, xw1)

    v2 = pl.pallas_call(
        functools.partial(_fused2_kernel, nj=nj, bj=_BJ),
        grid=(ni, nj),
        in_specs=[
            _rowblk(_BI, d),
            _full((d, n)),
            _full((n, d)),
            _rowblk(_BI, 1),
            _full((d, d)),
        ],
        out_specs=_rowblk(_BI, d),
        out_shape=jax.ShapeDtypeStruct((n, d), _BF),
        scratch_shapes=[pltpu.VMEM((_BI, d), f32)],
        compiler_params=params2,
    )(hn2, hnT2, mcat2, dinv2, W_ms2)

    jk2 = sjk_pass(hn2, hnT2, v2, dinv2)

    return jnp.concatenate([jk0, jk1, jk2], axis=-1)


# speedup vs baseline: 1.5368x; 1.1528x over previous
import functools

import jax
import jax.numpy as jnp
from jax.experimental import pallas as pl
from jax.experimental.pallas import tpu as pltpu

_SKIP = 0.5

_BI = 1024
_BJ = 2048
_BIA = 1024

_BF = jnp.bfloat16


def _dot(a, b):
    return jax.lax.dot_general(
        a, b, (((1,), (0,)), ((), ())),
        preferred_element_type=jnp.float32)


def _dot_sm(a, b):
    return jax.lax.dot_general(
        a.astype(_BF), b.astype(_BF), (((1,), (0,)), ((), ())),
        preferred_element_type=jnp.float32)


def _dinv_of(deg):
    return jnp.where(deg > 0, 1.0 / jnp.sqrt(deg + 1e-12), 0.0)


def _dinv0_kernel(a_ref, x_ref, w1_ref, wi_ref,
                  dinv_ref, abf_ref, xw1_ref, m0_ref,
                  acc_ref, *, nj, bj, d):
    j = pl.program_id(1)

    @pl.when(j == 0)
    def _():
        acc_ref[...] = jnp.zeros_like(acc_ref)

    abf = a_ref[...].astype(_BF)
    abf_ref[...] = abf
    ones = jnp.ones((bj, d), dtype=_BF)
    acc_ref[...] += _dot(abf, ones)

    @pl.when(j == nj - 1)
    def _():
        d0 = _dinv_of(acc_ref[:, :1])
        dinv_ref[...] = d0
        xw1 = _dot_sm(x_ref[...], w1_ref[...])
        xwi = _dot_sm(x_ref[...], wi_ref[...])
        xw1_ref[...] = xw1.astype(_BF)
        m0_ref[:, :d] = (d0 * xw1).astype(_BF)
        m0_ref[:, d:] = (d0 * xwi).astype(_BF)


def _init_kernel(ah_ref, m0_ref, dinv0_ref, w2_ref, wg_ref,
                 hn_ref, hnT_ref, g_ref, gv0_ref,
                 acc_ref, *, nj, bj, d):
    j = pl.program_id(1)

    @pl.when(j == 0)
    def _():
        acc_ref[...] = jnp.zeros_like(acc_ref)

    acc_ref[...] += _dot(ah_ref[...], m0_ref[pl.ds(j * bj, bj), :])

    @pl.when(j == nj - 1)
    def _():
        d0 = dinv0_ref[...]
        p = acc_ref[...] * d0
        h_c0 = jnp.maximum(p[:, :d], 0.0)
        h = jnp.maximum(p[:, d:], 0.0)
        nrm = jnp.sqrt(jnp.sum(h * h, axis=1, keepdims=True)) + 1e-8
        hn = (h / nrm).astype(_BF)
        hn_ref[...] = hn
        hnT_ref[...] = hn.T
        g = _dot_sm(h, wg_ref[...])
        g_ref[...] = g.astype(_BF)
        gv0_ref[:, :d] = (d0 * g).astype(_BF)
        v0 = d0 * _dot_sm(h_c0, w2_ref[...])
        gv0_ref[:, d:] = v0.astype(_BF)


def _deg_kernel(hn_ref, hnT_ref, g_ref, xw1_ref, dinv_ref, mcat_ref,
                acc_ref, *, nj, bj, d):
    j = pl.program_id(1)

    @pl.when(j == 0)
    def _():
        acc_ref[...] = jnp.zeros_like(acc_ref)

    s = jnp.maximum(_dot(hn_ref[...], hnT_ref[:, pl.ds(j * bj, bj)]), 0.0)
    acc_ref[...] += jnp.sum(s, axis=1, keepdims=True)

    @pl.when(j == nj - 1)
    def _():
        dinv = _dinv_of(acc_ref[...])
        dinv_ref[...] = dinv
        mcat_ref[:, :d] = (dinv * g_ref[...].astype(jnp.float32)).astype(_BF)
        mcat_ref[:, d:] = (dinv * xw1_ref[...].astype(jnp.float32)).astype(_BF)


def _fused_kernel(hn_ref, hnT_ref, mcat_ref, ah_ref, gv0_ref, dinv_ref,
                  dinv0_ref, w2_ref, wgn_ref, mask_ref,
                  hn2_ref, hnT2_ref, g2_ref, v_ref, jk0_ref,
                  accs_ref, accc_ref, *, nj, bj, d):
    j = pl.program_id(1)

    @pl.when(j == 0)
    def _():
        accs_ref[...] = jnp.zeros_like(accs_ref)
        accc_ref[...] = jnp.zeros_like(accc_ref)

    s = jnp.maximum(_dot(hn_ref[...], hnT_ref[:, pl.ds(j * bj, bj)]), 0.0)
    accs_ref[...] += _dot(s.astype(_BF), mcat_ref[pl.ds(j * bj, bj), :])
    accc_ref[...] += _dot(ah_ref[...], gv0_ref[pl.ds(j * bj, bj), :])

    @pl.when(j == nj - 1)
    def _():
        dinv = dinv_ref[...]
        d0 = dinv0_ref[...]
        h_new = jnp.maximum(
            _SKIP * d0 * accc_ref[:, :d] + (1.0 - _SKIP) * dinv * accs_ref[:, :d],
            0.0)
        h2 = jnp.maximum(dinv * accs_ref[:, d:], 0.0)
        jk0_ref[...] = mask_ref[...] * d0 * accc_ref[:, d:]
        v_ref[...] = (dinv * _dot_sm(h2, w2_ref[...])).astype(_BF)
        nrm = jnp.sqrt(jnp.sum(h_new * h_new, axis=1, keepdims=True)) + 1e-8
        hn2 = (h_new / nrm).astype(_BF)
        hn2_ref[...] = hn2
        hnT2_ref[...] = hn2.T
        g2_ref[...] = _dot_sm(h_new, wgn_ref[...]).astype(_BF)


def _fused2_kernel(hn_ref, hnT_ref, m_ref, dinv_ref, w2_ref, v_ref,
                   acc_ref, *, nj, bj):
    j = pl.program_id(1)

    @pl.when(j == 0)
    def _():
        acc_ref[...] = jnp.zeros_like(acc_ref)

    s = jnp.maximum(_dot(hn_ref[...], hnT_ref[:, pl.ds(j * bj, bj)]), 0.0)
    acc_ref[...] += _dot(s.astype(_BF), m_ref[pl.ds(j * bj, bj), :])

    @pl.when(j == nj - 1)
    def _():
        dinv = dinv_ref[...]
        h2 = jnp.maximum(dinv * acc_ref[...], 0.0)
        v_ref[...] = (dinv * _dot_sm(h2, w2_ref[...])).astype(_BF)


def _sjk_deg_kernel(hn_ref, hnT_ref, v_ref, dinv_ref, mask_ref,
                    hn2_ref, hnT2_ref, xw1_ref,
                    jk_ref, dinv2_ref, mcat2_ref,
                    acc_ref, deg_ref, *, nj, bj):
    j = pl.program_id(1)

    @pl.when(j == 0)
    def _():
        acc_ref[...] = jnp.zeros_like(acc_ref)
        deg_ref[...] = jnp.zeros_like(deg_ref)

    s = jnp.maximum(_dot(hn_ref[...], hnT_ref[:, pl.ds(j * bj, bj)]), 0.0)
    acc_ref[...] += _dot(s.astype(_BF), v_ref[pl.ds(j * bj, bj), :])
    s2 = jnp.maximum(_dot(hn2_ref[...], hnT2_ref[:, pl.ds(j * bj, bj)]), 0.0)
    deg_ref[...] += jnp.sum(s2, axis=1, keepdims=True)

    @pl.when(j == nj - 1)
    def _():
        jk_ref[...] = mask_ref[...] * dinv_ref[...] * acc_ref[...]
        dinv2 = _dinv_of(deg_ref[...])
        dinv2_ref[...] = dinv2
        mcat2_ref[...] = (dinv2 * xw1_ref[...].astype(jnp.float32)).astype(_BF)


def _sjk_kernel(hn_ref, hnT_ref, v_ref, dinv_ref, mask_ref, out_ref,
                acc_ref, *, nj, bj):
    j = pl.program_id(1)

    @pl.when(j == 0)
    def _():
        acc_ref[...] = jnp.zeros_like(acc_ref)

    s = jnp.maximum(_dot(hn_ref[...], hnT_ref[:, pl.ds(j * bj, bj)]), 0.0)
    acc_ref[...] += _dot(s.astype(_BF), v_ref[pl.ds(j * bj, bj), :])

    @pl.when(j == nj - 1)
    def _():
        out_ref[...] = mask_ref[...] * dinv_ref[...] * acc_ref[...]


def _full(shape):
    return pl.BlockSpec(shape, lambda i, j: (0, 0))


def _rowblk(b, w):
    return pl.BlockSpec((b, w), lambda i, j: (i, 0))


def kernel(X, initial_adj, minibatch_mask, W_ms1, W_ms2, W_init, W_gsl0, W_gsl1):
    n, d = X.shape
    f32 = jnp.float32
    maskf = minibatch_mask.astype(f32).reshape(n, 1)

    ni, nj = n // _BI, n // _BJ
    nia = n // _BIA
    params2 = pltpu.CompilerParams(
        dimension_semantics=("parallel", "arbitrary"))

    dinv0, a0bf, xw1, m0 = pl.pallas_call(
        functools.partial(_dinv0_kernel, nj=nia, bj=_BIA, d=d),
        grid=(nia, nia),
        in_specs=[pl.BlockSpec((_BIA, _BIA), lambda i, j: (i, j)),
                  _rowblk(_BIA, d),
                  _full((d, d)),
                  _full((d, d))],
        out_specs=[_rowblk(_BIA, 1),
                   pl.BlockSpec((_BIA, _BIA), lambda i, j: (i, j)),
                   _rowblk(_BIA, d),
                   _rowblk(_BIA, 2 * d)],
        out_shape=[jax.ShapeDtypeStruct((n, 1), f32),
                   jax.ShapeDtypeStruct((n, n), _BF),
                   jax.ShapeDtypeStruct((n, d), _BF),
                   jax.ShapeDtypeStruct((n, 2 * d), _BF)],
        scratch_shapes=[pltpu.VMEM((_BIA, d), f32)],
        compiler_params=params2,
    )(initial_adj, X, W_ms1, W_init)

    hn, hnT, g, gv0 = pl.pallas_call(
        functools.partial(_init_kernel, nj=nj, bj=_BJ, d=d),
        grid=(ni, nj),
        in_specs=[
            pl.BlockSpec((_BI, _BJ), lambda i, j: (i, j)),
            _full((n, 2 * d)),
            _rowblk(_BI, 1),
            _full((d, d)),
            _full((d, d)),
        ],
        out_specs=[
            _rowblk(_BI, d),
            pl.BlockSpec((d, _BI), lambda i, j: (0, i)),
            _rowblk(_BI, d),
            _rowblk(_BI, 2 * d),
        ],
        out_shape=[jax.ShapeDtypeStruct((n, d), _BF),
                   jax.ShapeDtypeStruct((d, n), _BF),
                   jax.ShapeDtypeStruct((n, d), _BF),
                   jax.ShapeDtypeStruct((n, 2 * d), _BF)],
        scratch_shapes=[pltpu.VMEM((_BI, 2 * d), f32)],
        compiler_params=params2,
    )(a0bf, m0, dinv0, W_ms2, W_gsl0)

    def deg_pass(hn_c, hnT_c, g_c, xw1_c):
        width = 2 * d
        return pl.pallas_call(
            functools.partial(_deg_kernel, nj=nj, bj=_BJ, d=d),
            grid=(ni, nj),
            in_specs=[
                _rowblk(_BI, d),
                _full((d, n)),
                _rowblk(_BI, d),
                _rowblk(_BI, d),
            ],
            out_specs=[_rowblk(_BI, 1), _rowblk(_BI, width)],
            out_shape=[jax.ShapeDtypeStruct((n, 1), f32),
                       jax.ShapeDtypeStruct((n, width), _BF)],
            scratch_shapes=[pltpu.VMEM((_BI, 1), f32)],
            compiler_params=params2,
        )(hn_c, hnT_c, g_c, xw1_c)

    def sjk_pass(hn_c, hnT_c, v_c, dinv_c):
        return pl.pallas_call(
            functools.partial(_sjk_kernel, nj=nj, bj=_BJ),
            grid=(ni, nj),
            in_specs=[
                _rowblk(_BI, d),
                _full((d, n)),
                _full((n, d)),
                _rowblk(_BI, 1),
                _rowblk(_BI, 1),
            ],
            out_specs=_rowblk(_BI, d),
            out_shape=jax.ShapeDtypeStruct((n, d), f32),
            scratch_shapes=[pltpu.VMEM((_BI, d), f32)],
            compiler_params=params2,
        )(hn_c, hnT_c, v_c, dinv_c, maskf)

    dinv1, mcat1 = deg_pass(hn, hnT, g, xw1)

    hn2, hnT2, g2, v1, jk0 = pl.pallas_call(
        functools.partial(_fused_kernel, nj=nj, bj=_BJ, d=d),
        grid=(ni, nj),
        in_specs=[
            _rowblk(_BI, d),
            _full((d, n)),
            _full((n, 2 * d)),
            pl.BlockSpec((_BI, _BJ), lambda i, j: (i, j)),
            _full((n, 2 * d)),
            _rowblk(_BI, 1),
            _rowblk(_BI, 1),
            _full((d, d)),
            _full((d, d)),
            _rowblk(_BI, 1),
        ],
        out_specs=[
            _rowblk(_BI, d),
            pl.BlockSpec((d, _BI), lambda i, j: (0, i)),
            _rowblk(_BI, d),
            _rowblk(_BI, d),
            _rowblk(_BI, d),
        ],
        out_shape=[jax.ShapeDtypeStruct((n, d), _BF),
                   jax.ShapeDtypeStruct((d, n), _BF),
                   jax.ShapeDtypeStruct((n, d), _BF),
                   jax.ShapeDtypeStruct((n, d), _BF),
                   jax.ShapeDtypeStruct((n, d), f32)],
        scratch_shapes=[pltpu.VMEM((_BI, 2 * d), f32),
                        pltpu.VMEM((_BI, 2 * d), f32)],
        compiler_params=params2,
    )(hn, hnT, mcat1, a0bf, gv0, dinv1, dinv0, W_ms2, W_gsl1, maskf)

    jk1, dinv2, mcat2 = pl.pallas_call(
        functools.partial(_sjk_deg_kernel, nj=nj, bj=_BJ),
        grid=(ni, nj),
        in_specs=[
            _rowblk(_BI, d),
            _full((d, n)),
            _full((n, d)),
            _rowblk(_BI, 1),
            _rowblk(_BI, 1),
            _rowblk(_BI, d),
            _full((d, n)),
            _rowblk(_BI, d),
        ],
        out_specs=[_rowblk(_BI, d), _rowblk(_BI, 1), _rowblk(_BI, d)],
        out_shape=[jax.ShapeDtypeStruct((n, d), f32),
                   jax.ShapeDtypeStruct((n, 1), f32),
                   jax.ShapeDtypeStruct((n, d), _BF)],
        scratch_shapes=[pltpu.VMEM((_BI, d), f32), pltpu.VMEM((_BI, 1), f32)],
        compiler_params=params2,
    )(hn, hnT, v1, dinv1, maskf, hn2, hnT2, xw1)

    v2 = pl.pallas_call(
        functools.partial(_fused2_kernel, nj=nj, bj=_BJ),
        grid=(ni, nj),
        in_specs=[
            _rowblk(_BI, d),
            _full((d, n)),
            _full((n, d)),
            _rowblk(_BI, 1),
            _full((d, d)),
        ],
        out_specs=_rowblk(_BI, d),
        out_shape=jax.ShapeDtypeStruct((n, d), _BF),
        scratch_shapes=[pltpu.VMEM((_BI, d), f32)],
        compiler_params=params2,
    )(hn2, hnT2, mcat2, dinv2, W_ms2)

    jk2 = sjk_pass(hn2, hnT2, v2, dinv2)

    return jnp.concatenate([jk0, jk1, jk2], axis=-1)


# speedup vs baseline: 1.5470x; 1.0066x over previous
import functools

import jax
import jax.numpy as jnp
from jax.experimental import pallas as pl
from jax.experimental.pallas import tpu as pltpu

_SKIP = 0.5

_BI = 1024
_BJ = 2048
_BIA = 2048

_BF = jnp.bfloat16


def _dot(a, b):
    return jax.lax.dot_general(
        a, b, (((1,), (0,)), ((), ())),
        preferred_element_type=jnp.float32)


def _dot_sm(a, b):
    return jax.lax.dot_general(
        a.astype(_BF), b.astype(_BF), (((1,), (0,)), ((), ())),
        preferred_element_type=jnp.float32)


def _dinv_of(deg):
    return jnp.where(deg > 0, 1.0 / jnp.sqrt(deg + 1e-12), 0.0)


def _dinv0_kernel(a_ref, x_ref, w1_ref, wi_ref,
                  dinv_ref, abf_ref, xw1_ref, m0_ref,
                  acc_ref, *, nj, bj, d):
    j = pl.program_id(1)

    @pl.when(j == 0)
    def _():
        acc_ref[...] = jnp.zeros_like(acc_ref)

    abf = a_ref[...].astype(_BF)
    abf_ref[...] = abf
    ones = jnp.ones((bj, d), dtype=_BF)
    acc_ref[...] += _dot(abf, ones)

    @pl.when(j == nj - 1)
    def _():
        d0 = _dinv_of(acc_ref[:, :1])
        dinv_ref[...] = d0
        xw1 = _dot_sm(x_ref[...], w1_ref[...])
        xwi = _dot_sm(x_ref[...], wi_ref[...])
        xw1_ref[...] = xw1.astype(_BF)
        m0_ref[:, :d] = (d0 * xw1).astype(_BF)
        m0_ref[:, d:] = (d0 * xwi).astype(_BF)


def _init_kernel(ah_ref, m0_ref, dinv0_ref, w2_ref, wg_ref,
                 hn_ref, hnT_ref, g_ref, gv0_ref,
                 acc_ref, *, nj, bj, d):
    j = pl.program_id(1)

    @pl.when(j == 0)
    def _():
        acc_ref[...] = jnp.zeros_like(acc_ref)

    acc_ref[...] += _dot(ah_ref[...], m0_ref[pl.ds(j * bj, bj), :])

    @pl.when(j == nj - 1)
    def _():
        d0 = dinv0_ref[...]
        p = acc_ref[...] * d0
        h_c0 = jnp.maximum(p[:, :d], 0.0)
        h = jnp.maximum(p[:, d:], 0.0)
        nrm = jnp.sqrt(jnp.sum(h * h, axis=1, keepdims=True)) + 1e-8
        hn = (h / nrm).astype(_BF)
        hn_ref[...] = hn
        hnT_ref[...] = hn.T
        g = _dot_sm(h, wg_ref[...])
        g_ref[...] = g.astype(_BF)
        gv0_ref[:, :d] = (d0 * g).astype(_BF)
        v0 = d0 * _dot_sm(h_c0, w2_ref[...])
        gv0_ref[:, d:] = v0.astype(_BF)


def _deg_kernel(hn_ref, hnT_ref, g_ref, xw1_ref, dinv_ref, mcat_ref,
                acc_ref, *, nj, bj, d):
    j = pl.program_id(1)

    @pl.when(j == 0)
    def _():
        acc_ref[...] = jnp.zeros_like(acc_ref)

    s = jnp.maximum(_dot(hn_ref[...], hnT_ref[:, pl.ds(j * bj, bj)]), 0.0)
    acc_ref[...] += jnp.sum(s, axis=1, keepdims=True)

    @pl.when(j == nj - 1)
    def _():
        dinv = _dinv_of(acc_ref[...])
        dinv_ref[...] = dinv
        mcat_ref[:, :d] = (dinv * g_ref[...].astype(jnp.float32)).astype(_BF)
        mcat_ref[:, d:] = (dinv * xw1_ref[...].astype(jnp.float32)).astype(_BF)


def _fused_kernel(hn_ref, hnT_ref, mcat_ref, ah_ref, gv0_ref, dinv_ref,
                  dinv0_ref, w2_ref, wgn_ref, mask_ref,
                  hn2_ref, hnT2_ref, g2_ref, v_ref, jk0_ref,
                  accs_ref, accc_ref, *, nj, bj, d):
    j = pl.program_id(1)

    @pl.when(j == 0)
    def _():
        accs_ref[...] = jnp.zeros_like(accs_ref)
        accc_ref[...] = jnp.zeros_like(accc_ref)

    s = jnp.maximum(_dot(hn_ref[...], hnT_ref[:, pl.ds(j * bj, bj)]), 0.0)
    accs_ref[...] += _dot(s.astype(_BF), mcat_ref[pl.ds(j * bj, bj), :])
    accc_ref[...] += _dot(ah_ref[...], gv0_ref[pl.ds(j * bj, bj), :])

    @pl.when(j == nj - 1)
    def _():
        dinv = dinv_ref[...]
        d0 = dinv0_ref[...]
        h_new = jnp.maximum(
            _SKIP * d0 * accc_ref[:, :d] + (1.0 - _SKIP) * dinv * accs_ref[:, :d],
            0.0)
        h2 = jnp.maximum(dinv * accs_ref[:, d:], 0.0)
        jk0_ref[...] = mask_ref[...] * d0 * accc_ref[:, d:]
        v_ref[...] = (dinv * _dot_sm(h2, w2_ref[...])).astype(_BF)
        nrm = jnp.sqrt(jnp.sum(h_new * h_new, axis=1, keepdims=True)) + 1e-8
        hn2 = (h_new / nrm).astype(_BF)
        hn2_ref[...] = hn2
        hnT2_ref[...] = hn2.T
        g2_ref[...] = _dot_sm(h_new, wgn_ref[...]).astype(_BF)


def _fused2_kernel(hn_ref, hnT_ref, m_ref, dinv_ref, w2_ref, v_ref,
                   acc_ref, *, nj, bj):
    j = pl.program_id(1)

    @pl.when(j == 0)
    def _():
        acc_ref[...] = jnp.zeros_like(acc_ref)

    s = jnp.maximum(_dot(hn_ref[...], hnT_ref[:, pl.ds(j * bj, bj)]), 0.0)
    acc_ref[...] += _dot(s.astype(_BF), m_ref[pl.ds(j * bj, bj), :])

    @pl.when(j == nj - 1)
    def _():
        dinv = dinv_ref[...]
        h2 = jnp.maximum(dinv * acc_ref[...], 0.0)
        v_ref[...] = (dinv * _dot_sm(h2, w2_ref[...])).astype(_BF)


def _sjk_deg_kernel(hn_ref, hnT_ref, v_ref, dinv_ref, mask_ref,
                    hn2_ref, hnT2_ref, xw1_ref,
                    jk_ref, dinv2_ref, mcat2_ref,
                    acc_ref, deg_ref, *, nj, bj):
    j = pl.program_id(1)

    @pl.when(j == 0)
    def _():
        acc_ref[...] = jnp.zeros_like(acc_ref)
        deg_ref[...] = jnp.zeros_like(deg_ref)

    s = jnp.maximum(_dot(hn_ref[...], hnT_ref[:, pl.ds(j * bj, bj)]), 0.0)
    acc_ref[...] += _dot(s.astype(_BF), v_ref[pl.ds(j * bj, bj), :])
    s2 = jnp.maximum(_dot(hn2_ref[...], hnT2_ref[:, pl.ds(j * bj, bj)]), 0.0)
    deg_ref[...] += jnp.sum(s2, axis=1, keepdims=True)

    @pl.when(j == nj - 1)
    def _():
        jk_ref[...] = mask_ref[...] * dinv_ref[...] * acc_ref[...]
        dinv2 = _dinv_of(deg_ref[...])
        dinv2_ref[...] = dinv2
        mcat2_ref[...] = (dinv2 * xw1_ref[...].astype(jnp.float32)).astype(_BF)


def _sjk_kernel(hn_ref, hnT_ref, v_ref, dinv_ref, mask_ref, out_ref,
                acc_ref, *, nj, bj):
    j = pl.program_id(1)

    @pl.when(j == 0)
    def _():
        acc_ref[...] = jnp.zeros_like(acc_ref)

    s = jnp.maximum(_dot(hn_ref[...], hnT_ref[:, pl.ds(j * bj, bj)]), 0.0)
    acc_ref[...] += _dot(s.astype(_BF), v_ref[pl.ds(j * bj, bj), :])

    @pl.when(j == nj - 1)
    def _():
        out_ref[...] = mask_ref[...] * dinv_ref[...] * acc_ref[...]


def _full(shape):
    return pl.BlockSpec(shape, lambda i, j: (0, 0))


def _rowblk(b, w):
    return pl.BlockSpec((b, w), lambda i, j: (i, 0))


def kernel(X, initial_adj, minibatch_mask, W_ms1, W_ms2, W_init, W_gsl0, W_gsl1):
    n, d = X.shape
    f32 = jnp.float32
    maskf = minibatch_mask.astype(f32).reshape(n, 1)

    ni, nj = n // _BI, n // _BJ
    nia = n // _BIA
    params2 = pltpu.CompilerParams(
        dimension_semantics=("parallel", "arbitrary"))

    dinv0, a0bf, xw1, m0 = pl.pallas_call(
        functools.partial(_dinv0_kernel, nj=nia, bj=_BIA, d=d),
        grid=(nia, nia),
        in_specs=[pl.BlockSpec((_BIA, _BIA), lambda i, j: (i, j)),
                  _rowblk(_BIA, d),
                  _full((d, d)),
                  _full((d, d))],
        out_specs=[_rowblk(_BIA, 1),
                   pl.BlockSpec((_BIA, _BIA), lambda i, j: (i, j)),
                   _rowblk(_BIA, d),
                   _rowblk(_BIA, 2 * d)],
        out_shape=[jax.ShapeDtypeStruct((n, 1), f32),
                   jax.ShapeDtypeStruct((n, n), _BF),
                   jax.ShapeDtypeStruct((n, d), _BF),
                   jax.ShapeDtypeStruct((n, 2 * d), _BF)],
        scratch_shapes=[pltpu.VMEM((_BIA, d), f32)],
        compiler_params=params2,
    )(initial_adj, X, W_ms1, W_init)

    hn, hnT, g, gv0 = pl.pallas_call(
        functools.partial(_init_kernel, nj=nj, bj=_BJ, d=d),
        grid=(ni, nj),
        in_specs=[
            pl.BlockSpec((_BI, _BJ), lambda i, j: (i, j)),
            _full((n, 2 * d)),
            _rowblk(_BI, 1),
            _full((d, d)),
            _full((d, d)),
        ],
        out_specs=[
            _rowblk(_BI, d),
            pl.BlockSpec((d, _BI), lambda i, j: (0, i)),
            _rowblk(_BI, d),
            _rowblk(_BI, 2 * d),
        ],
        out_shape=[jax.ShapeDtypeStruct((n, d), _BF),
                   jax.ShapeDtypeStruct((d, n), _BF),
                   jax.ShapeDtypeStruct((n, d), _BF),
                   jax.ShapeDtypeStruct((n, 2 * d), _BF)],
        scratch_shapes=[pltpu.VMEM((_BI, 2 * d), f32)],
        compiler_params=params2,
    )(a0bf, m0, dinv0, W_ms2, W_gsl0)

    def deg_pass(hn_c, hnT_c, g_c, xw1_c):
        width = 2 * d
        return pl.pallas_call(
            functools.partial(_deg_kernel, nj=nj, bj=_BJ, d=d),
            grid=(ni, nj),
            in_specs=[
                _rowblk(_BI, d),
                _full((d, n)),
                _rowblk(_BI, d),
                _rowblk(_BI, d),
            ],
            out_specs=[_rowblk(_BI, 1), _rowblk(_BI, width)],
            out_shape=[jax.ShapeDtypeStruct((n, 1), f32),
                       jax.ShapeDtypeStruct((n, width), _BF)],
            scratch_shapes=[pltpu.VMEM((_BI, 1), f32)],
            compiler_params=params2,
        )(hn_c, hnT_c, g_c, xw1_c)

    def sjk_pass(hn_c, hnT_c, v_c, dinv_c):
        return pl.pallas_call(
            functools.partial(_sjk_kernel, nj=nj, bj=_BJ),
            grid=(ni, nj),
            in_specs=[
                _rowblk(_BI, d),
                _full((d, n)),
                _full((n, d)),
                _rowblk(_BI, 1),
                _rowblk(_BI, 1),
            ],
            out_specs=_rowblk(_BI, d),
            out_shape=jax.ShapeDtypeStruct((n, d), f32),
            scratch_shapes=[pltpu.VMEM((_BI, d), f32)],
            compiler_params=params2,
        )(hn_c, hnT_c, v_c, dinv_c, maskf)

    dinv1, mcat1 = deg_pass(hn, hnT, g, xw1)

    hn2, hnT2, g2, v1, jk0 = pl.pallas_call(
        functools.partial(_fused_kernel, nj=nj, bj=_BJ, d=d),
        grid=(ni, nj),
        in_specs=[
            _rowblk(_BI, d),
            _full((d, n)),
            _full((n, 2 * d)),
            pl.BlockSpec((_BI, _BJ), lambda i, j: (i, j)),
            _full((n, 2 * d)),
            _rowblk(_BI, 1),
            _rowblk(_BI, 1),
            _full((d, d)),
            _full((d, d)),
            _rowblk(_BI, 1),
        ],
        out_specs=[
            _rowblk(_BI, d),
            pl.BlockSpec((d, _BI), lambda i, j: (0, i)),
            _rowblk(_BI, d),
            _rowblk(_BI, d),
            _rowblk(_BI, d),
        ],
        out_shape=[jax.ShapeDtypeStruct((n, d), _BF),
                   jax.ShapeDtypeStruct((d, n), _BF),
                   jax.ShapeDtypeStruct((n, d), _BF),
                   jax.ShapeDtypeStruct((n, d), _BF),
                   jax.ShapeDtypeStruct((n, d), f32)],
        scratch_shapes=[pltpu.VMEM((_BI, 2 * d), f32),
                        pltpu.VMEM((_BI, 2 * d), f32)],
        compiler_params=params2,
    )(hn, hnT, mcat1, a0bf, gv0, dinv1, dinv0, W_ms2, W_gsl1, maskf)

    jk1, dinv2, mcat2 = pl.pallas_call(
        functools.partial(_sjk_deg_kernel, nj=nj, bj=_BJ),
        grid=(ni, nj),
        in_specs=[
            _rowblk(_BI, d),
            _full((d, n)),
            _full((n, d)),
            _rowblk(_BI, 1),
            _rowblk(_BI, 1),
            _rowblk(_BI, d),
            _full((d, n)),
            _rowblk(_BI, d),
        ],
        out_specs=[_rowblk(_BI, d), _rowblk(_BI, 1), _rowblk(_BI, d)],
        out_shape=[jax.ShapeDtypeStruct((n, d), f32),
                   jax.ShapeDtypeStruct((n, 1), f32),
                   jax.ShapeDtypeStruct((n, d), _BF)],
        scratch_shapes=[pltpu.VMEM((_BI, d), f32), pltpu.VMEM((_BI, 1), f32)],
        compiler_params=params2,
    )(hn, hnT, v1, dinv1, maskf, hn2, hnT2, xw1)

    v2 = pl.pallas_call(
        functools.partial(_fused2_kernel, nj=nj, bj=_BJ),
        grid=(ni, nj),
        in_specs=[
            _rowblk(_BI, d),
            _full((d, n)),
            _full((n, d)),
            _rowblk(_BI, 1),
            _full((d, d)),
        ],
        out_specs=_rowblk(_BI, d),
        out_shape=jax.ShapeDtypeStruct((n, d), _BF),
        scratch_shapes=[pltpu.VMEM((_BI, d), f32)],
        compiler_params=params2,
    )(hn2, hnT2, mcat2, dinv2, W_ms2)

    jk2 = sjk_pass(hn2, hnT2, v2, dinv2)

    return jnp.concatenate([jk0, jk1, jk2], axis=-1)
